# baseline (speedup 1.0000x reference)
import math

import jax
import jax.numpy as jnp
from jax.experimental import pallas as pl
from jax.experimental.pallas import tpu as pltpu

_EPS = 1e-12
_EPS2 = _EPS * _EPS

_TARGET_BLOCK_BYTES = 2 << 20
_MIN_STEPS = 8


def _cdiv(a, b):
    return -(-a // b)


def _l2_lane_kernel(x_ref, o_ref):
    xf = x_ref[...]
    if xf.dtype != jnp.float32:
        xf = xf.astype(jnp.float32)
    ss = jnp.sum(xf * xf, axis=-1, keepdims=True)
    o_ref[...] = (xf * jax.lax.rsqrt(jnp.maximum(ss, _EPS2))).astype(o_ref.dtype)


def _normalize_last(x2):
    rows, c = x2.shape
    itemsize = jnp.dtype(x2.dtype).itemsize

    tile_rows = max(8, min(rows, _TARGET_BLOCK_BYTES // (c * itemsize)) // 8 * 8)
    while _cdiv(rows, tile_rows) < _MIN_STEPS and tile_rows > 8:
        tile_rows = max(8, (tile_rows // 2) // 8 * 8)

    blk = tile_rows * c * itemsize
    grid = (_cdiv(rows, tile_rows),)

    return pl.pallas_call(
        _l2_lane_kernel,
        out_shape=jax.ShapeDtypeStruct((rows, c), x2.dtype),
        grid=grid,
        in_specs=[pl.BlockSpec((tile_rows, c), lambda i: (i, 0))],
        out_specs=pl.BlockSpec((tile_rows, c), lambda i: (i, 0)),
        compiler_params=pltpu.CompilerParams(
            dimension_semantics=("parallel",),
            vmem_limit_bytes=min(int(5 * blk) + (2 << 20), 48 << 20),
        ),
    )(x2)


def _l2_mid_kernel(x_ref, o_ref):
    xf = x_ref[...]
    if xf.dtype != jnp.float32:
        xf = xf.astype(jnp.float32)
    ss = jnp.sum(xf * xf, axis=1, keepdims=True)
    o_ref[...] = (xf * jax.lax.rsqrt(jnp.maximum(ss, _EPS2))).astype(o_ref.dtype)


def _normalize_mid(x3):
    lead, c, trail = x3.shape
    itemsize = jnp.dtype(x3.dtype).itemsize
    tile_t = trail if trail <= 128 else max(
        128, min(trail, _TARGET_BLOCK_BYTES // (c * itemsize)) // 128 * 128)
    tile_lead = max(1, min(lead, _TARGET_BLOCK_BYTES // (c * tile_t * itemsize)))
    grid = (_cdiv(lead, tile_lead), _cdiv(trail, tile_t))
    blk = tile_lead * c * tile_t * itemsize
    return pl.pallas_call(
        _l2_mid_kernel,
        out_shape=jax.ShapeDtypeStruct(x3.shape, x3.dtype),
        grid=grid,
        in_specs=[pl.BlockSpec((tile_lead, c, tile_t), lambda i, j: (i, 0, j))],
        out_specs=pl.BlockSpec((tile_lead, c, tile_t), lambda i, j: (i, 0, j)),
        compiler_params=pltpu.CompilerParams(
            dimension_semantics=("parallel", "parallel"),
            vmem_limit_bytes=min(int(5 * blk) + (2 << 20), 48 << 20),
        ),
    )(x3)


def kernel(x):
    shape = x.shape
    if x.ndim == 4 and shape[1] % 128 == 0:
        n, c, h, w = shape
        x2 = jnp.transpose(x, (0, 2, 3, 1)).reshape(n * h * w, c)
        y2 = _normalize_last(x2)
        return jnp.transpose(y2.reshape(n, h, w, c), (0, 3, 1, 2))
    lead, c = shape[0], shape[1]
    trail = math.prod(shape[2:]) if len(shape) > 2 else 1
    return _normalize_mid(x.reshape(lead, c, trail)).reshape(shape)


# speedup vs baseline: 1.2101x; 1.2101x over previous
import math

import jax
import jax.numpy as jnp
from jax.experimental import pallas as pl
from jax.experimental.pallas import tpu as pltpu

_EPS = 1e-12
_EPS2 = _EPS * _EPS

_TARGET_BLOCK_BYTES = 11 << 20
_MIN_STEPS = 4


def _cdiv(a, b):
    return -(-a // b)


def _l2_lane_kernel(x_ref, o_ref):
    xf = x_ref[...]
    if xf.dtype != jnp.float32:
        xf = xf.astype(jnp.float32)
    ss = jnp.sum(xf * xf, axis=-1, keepdims=True)
    o_ref[...] = (xf * jax.lax.rsqrt(jnp.maximum(ss, _EPS2))).astype(o_ref.dtype)


def _normalize_last(x2):
    rows, c = x2.shape
    itemsize = jnp.dtype(x2.dtype).itemsize

    tile_rows = max(8, min(rows, _TARGET_BLOCK_BYTES // (c * itemsize)) // 8 * 8)
    while _cdiv(rows, tile_rows) < _MIN_STEPS and tile_rows > 8:
        tile_rows = max(8, (tile_rows // 2) // 8 * 8)

    blk = tile_rows * c * itemsize
    grid = (_cdiv(rows, tile_rows),)

    return pl.pallas_call(
        _l2_lane_kernel,
        out_shape=jax.ShapeDtypeStruct((rows, c), x2.dtype),
        grid=grid,
        in_specs=[pl.BlockSpec((tile_rows, c), lambda i: (i, 0))],
        out_specs=pl.BlockSpec((tile_rows, c), lambda i: (i, 0)),
        compiler_params=pltpu.CompilerParams(
            dimension_semantics=("parallel",),
            vmem_limit_bytes=min(int(5 * blk) + (2 << 20), 48 << 20),
        ),
    )(x2)


def _l2_mid_kernel(x_ref, o_ref):
    xf = x_ref[...]
    if xf.dtype != jnp.float32:
        xf = xf.astype(jnp.float32)
    ss = jnp.sum(xf * xf, axis=1, keepdims=True)
    o_ref[...] = (xf * jax.lax.rsqrt(jnp.maximum(ss, _EPS2))).astype(o_ref.dtype)


def _normalize_mid(x3):
    lead, c, trail = x3.shape
    itemsize = jnp.dtype(x3.dtype).itemsize
    tile_t = trail if trail <= 128 else max(
        128, min(trail, _TARGET_BLOCK_BYTES // (c * itemsize)) // 128 * 128)
    tile_lead = max(1, min(lead, _TARGET_BLOCK_BYTES // (c * tile_t * itemsize)))
    grid = (_cdiv(lead, tile_lead), _cdiv(trail, tile_t))
    blk = tile_lead * c * tile_t * itemsize
    return pl.pallas_call(
        _l2_mid_kernel,
        out_shape=jax.ShapeDtypeStruct(x3.shape, x3.dtype),
        grid=grid,
        in_specs=[pl.BlockSpec((tile_lead, c, tile_t), lambda i, j: (i, 0, j))],
        out_specs=pl.BlockSpec((tile_lead, c, tile_t), lambda i, j: (i, 0, j)),
        compiler_params=pltpu.CompilerParams(
            dimension_semantics=("parallel", "parallel"),
            vmem_limit_bytes=min(int(5 * blk) + (2 << 20), 48 << 20),
        ),
    )(x3)


def kernel(x):
    shape = x.shape
    if x.ndim == 4 and shape[1] % 128 == 0:
        n, c, h, w = shape
        x2 = jnp.transpose(x, (0, 2, 3, 1)).reshape(n * h * w, c)
        y2 = _normalize_last(x2)
        return jnp.transpose(y2.reshape(n, h, w, c), (0, 3, 1, 2))
    lead, c = shape[0], shape[1]
    trail = math.prod(shape[2:]) if len(shape) > 2 else 1
    return _normalize_mid(x.reshape(lead, c, trail)).reshape(shape)
